# Initial kernel scaffold; baseline (speedup 1.0000x reference)
import jax
import jax.numpy as jnp
from jax import lax
import jax.experimental.pallas as pl
from jax.experimental.pallas import tpu as pltpu
from jax.experimental.pallas import tpu_sc as plsc

T = 4
N = 10000
E = 320000
D = 128
NPAD = 10240
NW = 32
EW_PER = E // NW
CHUNK = 80
NCH = EW_PER // CHUNK
ROWS_PER = NPAD // 16

_HI = jax.lax.Precision.HIGHEST


def _sc_deg_kernel(col_hbm, ew_hbm, deg_hbm, colb, ewb, degl):
    c = lax.axis_index("c")
    s = lax.axis_index("s")
    w = c * 16 + s
    zero16 = jnp.zeros((16,), jnp.float32)

    @pl.loop(0, T)
    def _per_t(t):
        @pl.loop(0, NPAD // 16)
        def _zero(i):
            degl[pl.ds(i * 16, 16)] = zero16

        pltpu.sync_copy(col_hbm.at[t, w], colb)
        pltpu.sync_copy(ew_hbm.at[t, w], ewb)

        @pl.loop(0, NCH)
        def _scatter(r):
            for k in range(CHUNK // 16):
                cv = colb[r, pl.ds(k * 16, 16)]
                ev = ewb[r, pl.ds(k * 16, 16)]
                plsc.addupdate_scatter(degl, [cv], ev)

        pltpu.sync_copy(degl, deg_hbm.at[t, w])


def _sc_deg(col_r, ew_r):
    mesh = plsc.VectorSubcoreMesh(core_axis_name="c", subcore_axis_name="s")
    return pl.kernel(
        _sc_deg_kernel,
        out_type=jax.ShapeDtypeStruct((T, NW, NPAD), jnp.float32),
        mesh=mesh,
        scratch_types=[
            pltpu.VMEM((NCH, CHUNK), jnp.int32),
            pltpu.VMEM((NCH, CHUNK), jnp.float32),
            pltpu.VMEM((NPAD,), jnp.float32),
        ],
        name="sc_deg",
    )(col_r, ew_r)


def _tc_prep_kernel(deg_ref, x_ref, y_ref, dib_ref):
    deg = jnp.sum(deg_ref[0], axis=0) + 1.0
    dinv = jnp.where(deg > 0, lax.rsqrt(deg), 0.0)
    dcol = lax.broadcast_in_dim(dinv, (dinv.shape[0], D), (0,))
    x = x_ref[0]
    y_ref[0] = dcol * x
    dib_ref[0] = dcol


def _tc_prep(deg_part, x_seq):
    BN = 1000
    grid = (T, N // BN)
    y, dib = pl.pallas_call(
        _tc_prep_kernel,
        grid=grid,
        in_specs=[
            pl.BlockSpec((1, NW, BN), lambda t, i: (t, 0, i)),
            pl.BlockSpec((1, BN, D), lambda t, i: (t, i, 0)),
        ],
        out_specs=[
            pl.BlockSpec((1, BN, D), lambda t, i: (t, i, 0)),
            pl.BlockSpec((1, BN, D), lambda t, i: (t, i, 0)),
        ],
        out_shape=[
            jax.ShapeDtypeStruct((T, NPAD, D), jnp.float32),
            jax.ShapeDtypeStruct((T, N, D), jnp.float32),
        ],
        name="tc_prep",
    )(deg_part, x_seq)
    return y, dib


def _sc_main_kernel(row_hbm, col_hbm, ew_hbm, y_hbm, acc_hbm,
                    rowb, colb, ewb, gbuf, zrow, acc_sp, sem):
    c = lax.axis_index("c")
    s = lax.axis_index("s")
    w = c * 16 + s
    zero16 = jnp.zeros((16,), jnp.float32)

    @pl.loop(0, CHUNK)
    def _zr(r):
        for k in range(D // 16):
            zrow[r, pl.ds(k * 16, 16)] = zero16

    @pl.loop(0, T)
    def _per_t(t):
        @pl.loop(0, ROWS_PER // CHUNK)
        def _za(i):
            pltpu.sync_copy(
                zrow, acc_sp.at[pl.ds(s * ROWS_PER + i * CHUNK, CHUNK)])

        pltpu.sync_copy(row_hbm.at[t, w], rowb)
        pltpu.sync_copy(col_hbm.at[t, w], colb)
        pltpu.sync_copy(ew_hbm.at[t, w], ewb)
        plsc.subcore_barrier()

        @pl.loop(0, NCH)
        def _chunk(j):
            pltpu.async_copy(y_hbm.at[rowb.at[j]], gbuf, sem).wait()

            @pl.loop(0, CHUNK)
            def _edge(e):
                evec = plsc.load_gather(
                    ewb, [jnp.full((16,), j, jnp.int32),
                          jnp.full((16,), e, jnp.int32)])
                for k in range(D // 16):
                    gbuf[e, pl.ds(k * 16, 16)] = (
                        gbuf[e, pl.ds(k * 16, 16)] * evec)

            pltpu.sync_copy(gbuf, acc_sp.at[colb.at[j]], add=True)

        plsc.subcore_barrier()
        pltpu.sync_copy(acc_sp.at[pl.ds(s * ROWS_PER, ROWS_PER)],
                        acc_hbm.at[c, t, pl.ds(s * ROWS_PER, ROWS_PER)])
        plsc.subcore_barrier()


def _sc_main(row_abs, col_r, ew_r, y_flat):
    mesh = plsc.VectorSubcoreMesh(core_axis_name="c", subcore_axis_name="s")
    return pl.kernel(
        _sc_main_kernel,
        out_type=jax.ShapeDtypeStruct((2, T, NPAD, D), jnp.float32),
        mesh=mesh,
        scratch_types=[
            pltpu.VMEM((NCH, CHUNK), jnp.int32),
            pltpu.VMEM((NCH, CHUNK), jnp.int32),
            pltpu.VMEM((NCH, CHUNK), jnp.float32),
            pltpu.VMEM((CHUNK, D), jnp.float32),
            pltpu.VMEM((CHUNK, D), jnp.float32),
            pltpu.VMEM_SHARED((NPAD, D), jnp.float32),
            pltpu.SemaphoreType.DMA,
        ],
        name="sc_main",
    )(row_abs, col_r, ew_r, y_flat)


def _tc_final_kernel(acc_ref, dib_ref, y_ref,
                     Wz_ref, Lz_ref, lbz_ref, bz_ref,
                     Wh_ref, Lh_ref, lbh_ref, bh_ref,
                     Wlin_ref, blin_ref, out_ref, hsum):
    t = pl.program_id(1)
    agg = dib_ref[0] * (acc_ref[0, 0] + acc_ref[1, 0] + y_ref[0])

    Lz1 = Lz_ref[:D, :]
    Lh1 = Lh_ref[:D, :]
    Mz = jnp.dot(Wz_ref[...], Lz1, precision=_HI)
    Mh = jnp.dot(Wh_ref[...], Lh1, precision=_HI)
    cz = jnp.dot(bz_ref[...], Lz1, precision=_HI) + lbz_ref[...]
    ch = jnp.dot(bh_ref[...], Lh1, precision=_HI) + lbh_ref[...]

    Z = jax.nn.sigmoid(jnp.dot(agg, Mz, precision=_HI) + cz)
    Ht = jnp.tanh(jnp.dot(agg, Mh, precision=_HI) + ch)
    Hn = (1.0 - Z) * Ht

    @pl.when(t == 0)
    def _():
        hsum[...] = Hn

    @pl.when(t > 0)
    def _():
        hsum[...] = hsum[...] + Hn

    @pl.when(t == T - 1)
    def _():
        out_ref[...] = (jnp.dot(hsum[...], Wlin_ref[...] * (1.0 / T),
                                precision=_HI) + blin_ref[...])


def _tc_final(acc, dib, y, W_z, L_z, lb_z, b_z, W_h, L_h, lb_h, b_h,
              W_lin, b_lin):
    BN = 1000
    grid = (N // BN, T)

    def full(shape):
        return pl.BlockSpec(shape, lambda i, t: tuple(0 for _ in shape))

    return pl.pallas_call(
        _tc_final_kernel,
        grid=grid,
        in_specs=[
            pl.BlockSpec((2, 1, BN, D), lambda i, t: (0, t, i, 0)),
            pl.BlockSpec((1, BN, D), lambda i, t: (t, i, 0)),
            pl.BlockSpec((1, BN, D), lambda i, t: (t, i, 0)),
            full((D, D)), full((2 * D, D)), full((1, D)), full((1, D)),
            full((D, D)), full((2 * D, D)), full((1, D)), full((1, D)),
            full((D, D)), full((1, D)),
        ],
        out_specs=pl.BlockSpec((BN, D), lambda i, t: (i, 0)),
        out_shape=jax.ShapeDtypeStruct((N, D), jnp.float32),
        scratch_shapes=[pltpu.VMEM((BN, D), jnp.float32)],
        name="tc_final",
    )(acc, dib, y, W_z, L_z, lb_z, b_z, W_h, L_h, lb_h, b_h, W_lin, b_lin)


def kernel(x_seq, edge_index_seq, edge_attr_seq, W_z, b_z, W_r, b_r, W_h, b_h,
           L_z, lb_z, L_r, lb_r, L_h, lb_h, W_lin, b_lin):
    del W_r, b_r, L_r, lb_r

    row = edge_index_seq[:, 0, :].reshape(T, NW, NCH, CHUNK)
    col = edge_index_seq[:, 1, :].reshape(T, NW, NCH, CHUNK)
    ew = edge_attr_seq[:, :, 0].reshape(T, NW, NCH, CHUNK)

    deg_part = _sc_deg(col, ew)
    y, dib = _tc_prep(deg_part, x_seq)

    toff = (jnp.arange(T, dtype=jnp.int32) * NPAD).reshape(T, 1, 1, 1)
    row_abs = row + toff
    acc = _sc_main(row_abs, col, ew, y.reshape(T * NPAD, D))

    out = _tc_final(acc[:, :, :N, :], dib, y[:, :N, :],
                    W_z, L_z, lb_z.reshape(1, D), b_z.reshape(1, D),
                    W_h, L_h, lb_h.reshape(1, D), b_h.reshape(1, D),
                    W_lin, b_lin.reshape(1, D))
    return out


# speedup vs baseline: 25.9181x; 25.9181x over previous
import jax
import jax.numpy as jnp
from jax import lax
import jax.experimental.pallas as pl
from jax.experimental.pallas import tpu as pltpu
from jax.experimental.pallas import tpu_sc as plsc

T = 4
N = 10000
E = 320000
D = 128
NPAD = 10240
NW = 32
EW_PER = E // NW
CHUNK = 80
NCH = EW_PER // CHUNK
ROWS_PER = NPAD // 16

_HI = jax.lax.Precision.HIGHEST


def _sc_deg_kernel(col_hbm, ew_hbm, deg_hbm, colb, ewb, degl):
    c = lax.axis_index("c")
    s = lax.axis_index("s")
    w = c * 16 + s
    zero16 = jnp.zeros((16,), jnp.float32)

    @pl.loop(0, T)
    def _per_t(t):
        @pl.loop(0, NPAD // 16)
        def _zero(i):
            degl[pl.ds(i * 16, 16)] = zero16

        @pl.loop(0, NSUB)
        def _sub(b):
            pltpu.sync_copy(col_hbm.at[t, w, b], colb)
            pltpu.sync_copy(ew_hbm.at[t, w, b], ewb)

            @pl.loop(0, SUB)
            def _scatter(r):
                for k in range(CHUNK // 16):
                    cv = colb[r, pl.ds(k * 16, 16)]
                    ev = ewb[r, pl.ds(k * 16, 16)]
                    plsc.addupdate_scatter(degl, [cv], ev)

        pltpu.sync_copy(degl, deg_hbm.at[t, w])


def _sc_deg(col_r, ew_r):
    mesh = plsc.VectorSubcoreMesh(core_axis_name="c", subcore_axis_name="s")
    return pl.kernel(
        _sc_deg_kernel,
        out_type=jax.ShapeDtypeStruct((T, NW, NPAD), jnp.float32),
        mesh=mesh,
        scratch_types=[
            pltpu.VMEM((SUB, CHUNK), jnp.int32),
            pltpu.VMEM((SUB, CHUNK), jnp.float32),
            pltpu.VMEM((NPAD,), jnp.float32),
        ],
        compiler_params=pltpu.CompilerParams(needs_layout_passes=False),
        name="sc_deg",
    )(col_r, ew_r)


def _tc_prep_kernel(deg_ref, x_ref, y_ref, dib_ref):
    deg = jnp.sum(deg_ref[0], axis=0) + 1.0
    dinv = jnp.where(deg > 0, lax.rsqrt(deg), 0.0)
    dcol = lax.broadcast_in_dim(dinv, (dinv.shape[0], D), (0,))
    x = x_ref[0]
    y_ref[0] = dcol * x
    dib_ref[0] = dcol


def _tc_prep(deg_part, x_pad):
    BN = 1024
    grid = (T, NPAD // BN)
    y, dib = pl.pallas_call(
        _tc_prep_kernel,
        grid=grid,
        in_specs=[
            pl.BlockSpec((1, NW, BN), lambda t, i: (t, 0, i)),
            pl.BlockSpec((1, BN, D), lambda t, i: (t, i, 0)),
        ],
        out_specs=[
            pl.BlockSpec((1, BN, D), lambda t, i: (t, i, 0)),
            pl.BlockSpec((1, BN, D), lambda t, i: (t, i, 0)),
        ],
        out_shape=[
            jax.ShapeDtypeStruct((T, NPAD, D), jnp.float32),
            jax.ShapeDtypeStruct((T, NPAD, D), jnp.float32),
        ],
        name="tc_prep",
    )(deg_part, x_pad)
    return y, dib


SUB = 25
NSUB = NCH // SUB


def _sc_main_kernel(row_hbm, col_hbm, ew_hbm, y_hbm, acc_hbm,
                    rowb, colb, ewb, gbuf, zrow, acc_sp, sem):
    c = lax.axis_index("c")
    s = lax.axis_index("s")
    w = c * 16 + s
    zero16 = jnp.zeros((16,), jnp.float32)

    @pl.loop(0, 16)
    def _zr(r):
        for k in range(D // 16):
            zrow[r, pl.ds(k * 16, 16)] = zero16

    @pl.loop(0, T)
    def _per_t(t):
        @pl.loop(0, ROWS_PER // 16)
        def _za(i):
            pltpu.sync_copy(
                zrow, acc_sp.at[pl.ds(s * ROWS_PER + i * 16, 16)])
        plsc.subcore_barrier()

        @pl.loop(0, NSUB)
        def _sub(b):
            pltpu.sync_copy(row_hbm.at[t, w, b], rowb)
            pltpu.sync_copy(col_hbm.at[t, w, b], colb)
            pltpu.sync_copy(ew_hbm.at[t, w, b], ewb)

            @pl.loop(0, SUB)
            def _chunk(j):
                pltpu.async_copy(y_hbm.at[rowb.at[j]], gbuf, sem).wait()

                @pl.loop(0, CHUNK)
                def _edge(e):
                    evec = plsc.load_gather(
                        ewb, [jnp.full((16,), j, jnp.int32),
                              jnp.full((16,), e, jnp.int32)])
                    for k in range(D // 16):
                        gbuf[e, pl.ds(k * 16, 16)] = (
                            gbuf[e, pl.ds(k * 16, 16)] * evec)

                pltpu.sync_copy(gbuf, acc_sp.at[colb.at[j]], add=True)

        plsc.subcore_barrier()
        pltpu.sync_copy(acc_sp.at[pl.ds(s * ROWS_PER, ROWS_PER)],
                        acc_hbm.at[c, t, pl.ds(s * ROWS_PER, ROWS_PER)])
        plsc.subcore_barrier()


def _sc_main(row_abs, col_r, ew_r, y_flat):
    mesh = plsc.VectorSubcoreMesh(core_axis_name="c", subcore_axis_name="s")
    return pl.kernel(
        _sc_main_kernel,
        out_type=jax.ShapeDtypeStruct((2, T, NPAD, D), jnp.float32),
        mesh=mesh,
        scratch_types=[
            pltpu.VMEM((SUB, CHUNK), jnp.int32),
            pltpu.VMEM((SUB, CHUNK), jnp.int32),
            pltpu.VMEM((SUB, CHUNK), jnp.float32),
            pltpu.VMEM((CHUNK, D), jnp.float32),
            pltpu.VMEM((16, D), jnp.float32),
            pltpu.VMEM_SHARED((NPAD, D), jnp.float32),
            pltpu.SemaphoreType.DMA,
        ],
        compiler_params=pltpu.CompilerParams(needs_layout_passes=False),
        name="sc_main",
    )(row_abs, col_r, ew_r, y_flat)


def _tc_final_kernel(acc_ref, dib_ref, y_ref,
                     Wz_ref, Lz_ref, lbz_ref, bz_ref,
                     Wh_ref, Lh_ref, lbh_ref, bh_ref,
                     Wlin_ref, blin_ref, out_ref, hsum):
    t = pl.program_id(1)
    agg = dib_ref[0] * (acc_ref[0, 0] + acc_ref[1, 0] + y_ref[0])

    Lz1 = Lz_ref[:D, :]
    Lh1 = Lh_ref[:D, :]
    Mz = jnp.dot(Wz_ref[...], Lz1, precision=_HI)
    Mh = jnp.dot(Wh_ref[...], Lh1, precision=_HI)
    cz = jnp.dot(bz_ref[...], Lz1, precision=_HI) + lbz_ref[...]
    ch = jnp.dot(bh_ref[...], Lh1, precision=_HI) + lbh_ref[...]

    Z = jax.nn.sigmoid(jnp.dot(agg, Mz, precision=_HI) + cz)
    Ht = jnp.tanh(jnp.dot(agg, Mh, precision=_HI) + ch)
    Hn = (1.0 - Z) * Ht

    @pl.when(t == 0)
    def _():
        hsum[...] = Hn

    @pl.when(t > 0)
    def _():
        hsum[...] = hsum[...] + Hn

    @pl.when(t == T - 1)
    def _():
        out_ref[...] = (jnp.dot(hsum[...], Wlin_ref[...] * (1.0 / T),
                                precision=_HI) + blin_ref[...])


def _tc_final(acc, dib, y, W_z, L_z, lb_z, b_z, W_h, L_h, lb_h, b_h,
              W_lin, b_lin):
    BN = 1024
    grid = (NPAD // BN, T)

    def full(shape):
        return pl.BlockSpec(shape, lambda i, t: tuple(0 for _ in shape))

    return pl.pallas_call(
        _tc_final_kernel,
        grid=grid,
        in_specs=[
            pl.BlockSpec((2, 1, BN, D), lambda i, t: (0, t, i, 0)),
            pl.BlockSpec((1, BN, D), lambda i, t: (t, i, 0)),
            pl.BlockSpec((1, BN, D), lambda i, t: (t, i, 0)),
            full((D, D)), full((2 * D, D)), full((1, D)), full((1, D)),
            full((D, D)), full((2 * D, D)), full((1, D)), full((1, D)),
            full((D, D)), full((1, D)),
        ],
        out_specs=pl.BlockSpec((BN, D), lambda i, t: (i, 0)),
        out_shape=jax.ShapeDtypeStruct((NPAD, D), jnp.float32),
        scratch_shapes=[pltpu.VMEM((BN, D), jnp.float32)],
        name="tc_final",
    )(acc, dib, y, W_z, L_z, lb_z, b_z, W_h, L_h, lb_h, b_h, W_lin, b_lin)


def kernel(x_seq, edge_index_seq, edge_attr_seq, W_z, b_z, W_r, b_r, W_h, b_h,
           L_z, lb_z, L_r, lb_r, L_h, lb_h, W_lin, b_lin):
    del W_r, b_r, L_r, lb_r

    row = edge_index_seq[:, 0, :].reshape(T, NW, NSUB, SUB, CHUNK)
    col = edge_index_seq[:, 1, :].reshape(T, NW, NSUB, SUB, CHUNK)
    ew = edge_attr_seq[:, :, 0].reshape(T, NW, NSUB, SUB, CHUNK)

    deg_part = _sc_deg(col, ew)
    x_pad = jnp.pad(x_seq, ((0, 0), (0, NPAD - N), (0, 0)))
    y, dib = _tc_prep(deg_part, x_pad)

    toff = (jnp.arange(T, dtype=jnp.int32) * NPAD).reshape(T, 1, 1, 1, 1)
    row_abs = row + toff
    acc = _sc_main(row_abs, col, ew, y.reshape(T * NPAD, D))

    out = _tc_final(acc, dib, y,
                    W_z, L_z, lb_z.reshape(1, D), b_z.reshape(1, D),
                    W_h, L_h, lb_h.reshape(1, D), b_h.reshape(1, D),
                    W_lin, b_lin.reshape(1, D))
    return out[:N]


# speedup vs baseline: 40.5631x; 1.5650x over previous
import jax
import jax.numpy as jnp
from jax import lax
import jax.experimental.pallas as pl
from jax.experimental.pallas import tpu as pltpu
from jax.experimental.pallas import tpu_sc as plsc

T = 4
N = 10000
E = 320000
D = 128
NPAD = 10240
NW = 32
EW_PER = E // NW
ROWS_PER = NPAD // 16

CHUNK_D = 80
SUB_D = 25
NSUB_D = EW_PER // (SUB_D * CHUNK_D)

CHUNK = 100
NCH = EW_PER // CHUNK
SUB = 20
NSUB = NCH // SUB

_HI = jax.lax.Precision.HIGHEST


def _sc_deg_kernel(col_hbm, ew_hbm, deg_hbm, colb, ewb, degl):
    c = lax.axis_index("c")
    s = lax.axis_index("s")
    w = c * 16 + s
    zero16 = jnp.zeros((16,), jnp.float32)

    @pl.loop(0, T)
    def _per_t(t):
        @pl.loop(0, NPAD // 16)
        def _zero(i):
            degl[pl.ds(i * 16, 16)] = zero16

        @pl.loop(0, NSUB_D)
        def _sub(b):
            pltpu.sync_copy(col_hbm.at[t, w, b], colb)
            pltpu.sync_copy(ew_hbm.at[t, w, b], ewb)

            @pl.loop(0, SUB_D)
            def _scatter(r):
                for k in range(CHUNK_D // 16):
                    cv = colb[r, pl.ds(k * 16, 16)]
                    ev = ewb[r, pl.ds(k * 16, 16)]
                    plsc.addupdate_scatter(degl, [cv], ev)

        pltpu.sync_copy(degl, deg_hbm.at[t, w])


def _sc_deg(col_r, ew_r):
    mesh = plsc.VectorSubcoreMesh(core_axis_name="c", subcore_axis_name="s")
    return pl.kernel(
        _sc_deg_kernel,
        out_type=jax.ShapeDtypeStruct((T, NW, NPAD), jnp.float32),
        mesh=mesh,
        scratch_types=[
            pltpu.VMEM((SUB_D, CHUNK_D), jnp.int32),
            pltpu.VMEM((SUB_D, CHUNK_D), jnp.float32),
            pltpu.VMEM((NPAD,), jnp.float32),
        ],
        compiler_params=pltpu.CompilerParams(needs_layout_passes=False),
        name="sc_deg",
    )(col_r, ew_r)


def _tc_prep_kernel(deg_ref, x_ref, y_ref, dib_ref):
    deg = jnp.sum(deg_ref[0], axis=0) + 1.0
    dinv = jnp.where(deg > 0, lax.rsqrt(deg), 0.0)
    dcol = lax.broadcast_in_dim(dinv, (dinv.shape[0], D), (0,))
    x = x_ref[0]
    y_ref[0] = dcol * x
    dib_ref[0] = dcol


def _tc_prep(deg_part, x_pad):
    BN = 1024
    grid = (T, NPAD // BN)
    y, dib = pl.pallas_call(
        _tc_prep_kernel,
        grid=grid,
        in_specs=[
            pl.BlockSpec((1, NW, BN), lambda t, i: (t, 0, i)),
            pl.BlockSpec((1, BN, D), lambda t, i: (t, i, 0)),
        ],
        out_specs=[
            pl.BlockSpec((1, BN, D), lambda t, i: (t, i, 0)),
            pl.BlockSpec((1, BN, D), lambda t, i: (t, i, 0)),
        ],
        out_shape=[
            jax.ShapeDtypeStruct((T, NPAD, D), jnp.float32),
            jax.ShapeDtypeStruct((T, NPAD, D), jnp.float32),
        ],
        name="tc_prep",
    )(deg_part, x_pad)
    return y, dib


def _sc_main_kernel(row_hbm, col_hbm, ew_hbm, y_hbm, acc_hbm,
                    rowb, colb, ewb, gbufA, gbufB, zrow, acc_sp,
                    semA, semB):
    c = lax.axis_index("c")
    s = lax.axis_index("s")
    w = c * 16 + s
    zero16 = jnp.zeros((16,), jnp.float32)

    @pl.loop(0, 16)
    def _zr(r):
        for k in range(D // 16):
            zrow[r, pl.ds(k * 16, 16)] = zero16

    def _scale_scatter(gbuf, j_local):
        @pl.loop(0, CHUNK)
        def _edge(e):
            evec = plsc.load_gather(
                ewb, [jnp.full((16,), j_local, jnp.int32),
                      jnp.full((16,), e, jnp.int32)])
            for k in range(D // 16):
                gbuf[e, pl.ds(k * 16, 16)] = (
                    gbuf[e, pl.ds(k * 16, 16)] * evec)

        pltpu.sync_copy(gbuf, acc_sp.at[colb.at[j_local]], add=True)

    @pl.loop(0, T)
    def _per_t(t):
        @pl.loop(0, ROWS_PER // 16)
        def _za(i):
            pltpu.sync_copy(
                zrow, acc_sp.at[pl.ds(s * ROWS_PER + i * 16, 16)])
        plsc.subcore_barrier()

        pltpu.sync_copy(row_hbm.at[t, w], rowb)

        pltpu.async_copy(y_hbm.at[rowb.at[0]], gbufA, semA)

        @pl.loop(0, NSUB)
        def _sub(b):
            pltpu.sync_copy(col_hbm.at[t, w, b], colb)
            pltpu.sync_copy(ew_hbm.at[t, w, b], ewb)

            @pl.loop(0, SUB // 2)
            def _pair(jj):
                g0 = b * SUB + 2 * jj
                j0 = 2 * jj
                pltpu.async_copy(y_hbm.at[rowb.at[g0 + 1]], gbufB, semB)
                pltpu.make_async_copy(
                    y_hbm.at[rowb.at[g0]], gbufA, semA).wait()
                _scale_scatter(gbufA, j0)
                @pl.when(g0 + 2 < NCH)
                def _():
                    pltpu.async_copy(
                        y_hbm.at[rowb.at[g0 + 2]], gbufA, semA)
                pltpu.make_async_copy(
                    y_hbm.at[rowb.at[g0 + 1]], gbufB, semB).wait()
                _scale_scatter(gbufB, j0 + 1)

        plsc.subcore_barrier()
        pltpu.sync_copy(acc_sp.at[pl.ds(s * ROWS_PER, ROWS_PER)],
                        acc_hbm.at[c, t, pl.ds(s * ROWS_PER, ROWS_PER)])
        plsc.subcore_barrier()


def _sc_main(row_abs, col_r, ew_r, y_flat):
    mesh = plsc.VectorSubcoreMesh(core_axis_name="c", subcore_axis_name="s")
    return pl.kernel(
        _sc_main_kernel,
        out_type=jax.ShapeDtypeStruct((2, T, NPAD, D), jnp.float32),
        mesh=mesh,
        scratch_types=[
            pltpu.VMEM((NCH, CHUNK), jnp.int32),
            pltpu.VMEM((SUB, CHUNK), jnp.int32),
            pltpu.VMEM((SUB, CHUNK), jnp.float32),
            pltpu.VMEM((CHUNK, D), jnp.float32),
            pltpu.VMEM((CHUNK, D), jnp.float32),
            pltpu.VMEM((16, D), jnp.float32),
            pltpu.VMEM_SHARED((NPAD, D), jnp.float32),
            pltpu.SemaphoreType.DMA,
            pltpu.SemaphoreType.DMA,
        ],
        compiler_params=pltpu.CompilerParams(needs_layout_passes=False),
        name="sc_main",
    )(row_abs, col_r, ew_r, y_flat)


def _tc_final_kernel(acc_ref, dib_ref, y_ref,
                     Wz_ref, Lz_ref, lbz_ref, bz_ref,
                     Wh_ref, Lh_ref, lbh_ref, bh_ref,
                     Wlin_ref, blin_ref, out_ref, hsum):
    t = pl.program_id(1)
    agg = dib_ref[0] * (acc_ref[0, 0] + acc_ref[1, 0] + y_ref[0])

    Lz1 = Lz_ref[:D, :]
    Lh1 = Lh_ref[:D, :]
    Mz = jnp.dot(Wz_ref[...], Lz1, precision=_HI)
    Mh = jnp.dot(Wh_ref[...], Lh1, precision=_HI)
    cz = jnp.dot(bz_ref[...], Lz1, precision=_HI) + lbz_ref[...]
    ch = jnp.dot(bh_ref[...], Lh1, precision=_HI) + lbh_ref[...]

    Z = jax.nn.sigmoid(jnp.dot(agg, Mz, precision=_HI) + cz)
    Ht = jnp.tanh(jnp.dot(agg, Mh, precision=_HI) + ch)
    Hn = (1.0 - Z) * Ht

    @pl.when(t == 0)
    def _():
        hsum[...] = Hn

    @pl.when(t > 0)
    def _():
        hsum[...] = hsum[...] + Hn

    @pl.when(t == T - 1)
    def _():
        out_ref[...] = (jnp.dot(hsum[...], Wlin_ref[...] * (1.0 / T),
                                precision=_HI) + blin_ref[...])


def _tc_final(acc, dib, y, W_z, L_z, lb_z, b_z, W_h, L_h, lb_h, b_h,
              W_lin, b_lin):
    BN = 1024
    grid = (NPAD // BN, T)

    def full(shape):
        return pl.BlockSpec(shape, lambda i, t: tuple(0 for _ in shape))

    return pl.pallas_call(
        _tc_final_kernel,
        grid=grid,
        in_specs=[
            pl.BlockSpec((2, 1, BN, D), lambda i, t: (0, t, i, 0)),
            pl.BlockSpec((1, BN, D), lambda i, t: (t, i, 0)),
            pl.BlockSpec((1, BN, D), lambda i, t: (t, i, 0)),
            full((D, D)), full((2 * D, D)), full((1, D)), full((1, D)),
            full((D, D)), full((2 * D, D)), full((1, D)), full((1, D)),
            full((D, D)), full((1, D)),
        ],
        out_specs=pl.BlockSpec((BN, D), lambda i, t: (i, 0)),
        out_shape=jax.ShapeDtypeStruct((NPAD, D), jnp.float32),
        scratch_shapes=[pltpu.VMEM((BN, D), jnp.float32)],
        name="tc_final",
    )(acc, dib, y, W_z, L_z, lb_z, b_z, W_h, L_h, lb_h, b_h, W_lin, b_lin)


def kernel(x_seq, edge_index_seq, edge_attr_seq, W_z, b_z, W_r, b_r, W_h, b_h,
           L_z, lb_z, L_r, lb_r, L_h, lb_h, W_lin, b_lin):
    del W_r, b_r, L_r, lb_r

    col_d = edge_index_seq[:, 1, :].reshape(T, NW, NSUB_D, SUB_D, CHUNK_D)
    ew_d = edge_attr_seq[:, :, 0].reshape(T, NW, NSUB_D, SUB_D, CHUNK_D)

    row = edge_index_seq[:, 0, :].reshape(T, NW, NCH, CHUNK)
    col = edge_index_seq[:, 1, :].reshape(T, NW, NSUB, SUB, CHUNK)
    ew = edge_attr_seq[:, :, 0].reshape(T, NW, NSUB, SUB, CHUNK)

    deg_part = _sc_deg(col_d, ew_d)
    x_pad = jnp.pad(x_seq, ((0, 0), (0, NPAD - N), (0, 0)))
    y, dib = _tc_prep(deg_part, x_pad)

    toff = (jnp.arange(T, dtype=jnp.int32) * NPAD).reshape(T, 1, 1, 1)
    row_abs = row + toff
    acc = _sc_main(row_abs, col, ew, y.reshape(T * NPAD, D))

    out = _tc_final(acc, dib, y,
                    W_z, L_z, lb_z.reshape(1, D), b_z.reshape(1, D),
                    W_h, L_h, lb_h.reshape(1, D), b_h.reshape(1, D),
                    W_lin, b_lin.reshape(1, D))
    return out[:N]


# speedup vs baseline: 40.8843x; 1.0079x over previous
import jax
import jax.numpy as jnp
from jax import lax
import jax.experimental.pallas as pl
from jax.experimental.pallas import tpu as pltpu
from jax.experimental.pallas import tpu_sc as plsc

T = 4
N = 10000
E = 320000
D = 128
NPAD = 10240
NW = 32
EW_PER = E // NW
ROWS_PER = NPAD // 16

CHUNK_D = 80
SUB_D = 25
NSUB_D = EW_PER // (SUB_D * CHUNK_D)

CHUNK = 100
NCH = EW_PER // CHUNK
SUB = 20
NSUB = NCH // SUB

_HI = jax.lax.Precision.HIGHEST


def _sc_deg_kernel(col_hbm, ew_hbm, deg_hbm, colb, ewb, degl):
    c = lax.axis_index("c")
    s = lax.axis_index("s")
    w = c * 16 + s
    zero16 = jnp.zeros((16,), jnp.float32)

    @pl.loop(0, T)
    def _per_t(t):
        @pl.loop(0, NPAD // 16)
        def _zero(i):
            degl[pl.ds(i * 16, 16)] = zero16

        @pl.loop(0, NSUB_D)
        def _sub(b):
            pltpu.sync_copy(col_hbm.at[t, w, b], colb)
            pltpu.sync_copy(ew_hbm.at[t, w, b], ewb)

            @pl.loop(0, SUB_D)
            def _scatter(r):
                for k in range(CHUNK_D // 16):
                    cv = colb[r, pl.ds(k * 16, 16)]
                    ev = ewb[r, pl.ds(k * 16, 16)]
                    plsc.addupdate_scatter(degl, [cv], ev)

        pltpu.sync_copy(degl, deg_hbm.at[t, w])


def _sc_deg(col_r, ew_r):
    mesh = plsc.VectorSubcoreMesh(core_axis_name="c", subcore_axis_name="s")
    return pl.kernel(
        _sc_deg_kernel,
        out_type=jax.ShapeDtypeStruct((T, NW, NPAD), jnp.float32),
        mesh=mesh,
        scratch_types=[
            pltpu.VMEM((SUB_D, CHUNK_D), jnp.int32),
            pltpu.VMEM((SUB_D, CHUNK_D), jnp.float32),
            pltpu.VMEM((NPAD,), jnp.float32),
        ],
        compiler_params=pltpu.CompilerParams(needs_layout_passes=False),
        name="sc_deg",
    )(col_r, ew_r)


def _tc_prep_kernel(deg_ref, x_ref, y_ref, dib_ref):
    deg = jnp.sum(deg_ref[0], axis=0) + 1.0
    dinv = jnp.where(deg > 0, lax.rsqrt(deg), 0.0)
    dcol = lax.broadcast_in_dim(dinv, (dinv.shape[0], D), (0,))
    x = x_ref[0]
    y_ref[0] = dcol * x
    dib_ref[0] = dcol


def _tc_prep(deg_part, x_pad):
    BN = 1024
    grid = (T, NPAD // BN)
    y, dib = pl.pallas_call(
        _tc_prep_kernel,
        grid=grid,
        in_specs=[
            pl.BlockSpec((1, NW, BN), lambda t, i: (t, 0, i)),
            pl.BlockSpec((1, BN, D), lambda t, i: (t, i, 0)),
        ],
        out_specs=[
            pl.BlockSpec((1, BN, D), lambda t, i: (t, i, 0)),
            pl.BlockSpec((1, BN, D), lambda t, i: (t, i, 0)),
        ],
        out_shape=[
            jax.ShapeDtypeStruct((T, NPAD, D), jnp.float32),
            jax.ShapeDtypeStruct((T, NPAD, D), jnp.float32),
        ],
        name="tc_prep",
    )(deg_part, x_pad)
    return y, dib


def _sc_main_kernel(row_hbm, col_hbm, ew_hbm, y_hbm, acc_hbm,
                    rowb, colb, ewb, gbufA, gbufB, zrow, acc_sp,
                    semA, semB, semZ):
    c = lax.axis_index("c")
    s = lax.axis_index("s")
    w = c * 16 + s
    zero16 = jnp.zeros((16,), jnp.float32)

    @pl.loop(0, 16)
    def _zr(r):
        for k in range(D // 16):
            zrow[r, pl.ds(k * 16, 16)] = zero16

    def _scale_scatter(gbuf, j_local):
        base = j_local * CHUNK

        @pl.loop(0, CHUNK)
        def _edge(e):
            evec = plsc.load_gather(
                ewb, [jnp.full((16,), base + e, jnp.int32)])
            for k in range(D // 16):
                gbuf[e, pl.ds(k * 16, 16)] = (
                    gbuf[e, pl.ds(k * 16, 16)] * evec)

        pltpu.sync_copy(gbuf, acc_sp.at[colb.at[j_local]], add=True)

    def _zero_slice():
        @pl.loop(0, ROWS_PER // 16)
        def _za(i):
            pltpu.async_copy(
                zrow, acc_sp.at[pl.ds(s * ROWS_PER + i * 16, 16)], semZ)

        @pl.loop(0, ROWS_PER // 16)
        def _zw(i):
            pltpu.make_async_copy(
                zrow, acc_sp.at[pl.ds(s * ROWS_PER + i * 16, 16)],
                semZ).wait()

    _zero_slice()
    plsc.subcore_barrier()

    @pl.loop(0, T)
    def _per_t(t):
        pltpu.sync_copy(row_hbm.at[t, w], rowb)

        pltpu.async_copy(y_hbm.at[rowb.at[0]], gbufA, semA)

        @pl.loop(0, NSUB)
        def _sub(b):
            pltpu.sync_copy(col_hbm.at[t, w, b], colb)
            pltpu.sync_copy(ew_hbm.at[t, w, b], ewb)

            @pl.loop(0, SUB // 2)
            def _pair(jj):
                g0 = b * SUB + 2 * jj
                j0 = 2 * jj
                pltpu.async_copy(y_hbm.at[rowb.at[g0 + 1]], gbufB, semB)
                pltpu.make_async_copy(
                    y_hbm.at[rowb.at[g0]], gbufA, semA).wait()
                _scale_scatter(gbufA, j0)
                @pl.when(g0 + 2 < NCH)
                def _():
                    pltpu.async_copy(
                        y_hbm.at[rowb.at[g0 + 2]], gbufA, semA)
                pltpu.make_async_copy(
                    y_hbm.at[rowb.at[g0 + 1]], gbufB, semB).wait()
                _scale_scatter(gbufB, j0 + 1)

        plsc.subcore_barrier()
        pltpu.sync_copy(acc_sp.at[pl.ds(s * ROWS_PER, ROWS_PER)],
                        acc_hbm.at[c, t, pl.ds(s * ROWS_PER, ROWS_PER)])
        @pl.when(t + 1 < T)
        def _():
            _zero_slice()
        plsc.subcore_barrier()


def _sc_main(row_abs, col_r, ew_r, y_flat):
    mesh = plsc.VectorSubcoreMesh(core_axis_name="c", subcore_axis_name="s")
    return pl.kernel(
        _sc_main_kernel,
        out_type=jax.ShapeDtypeStruct((2, T, NPAD, D), jnp.float32),
        mesh=mesh,
        scratch_types=[
            pltpu.VMEM((NCH, CHUNK), jnp.int32),
            pltpu.VMEM((SUB, CHUNK), jnp.int32),
            pltpu.VMEM((SUB * CHUNK,), jnp.float32),
            pltpu.VMEM((CHUNK, D), jnp.float32),
            pltpu.VMEM((CHUNK, D), jnp.float32),
            pltpu.VMEM((16, D), jnp.float32),
            pltpu.VMEM_SHARED((NPAD, D), jnp.float32),
            pltpu.SemaphoreType.DMA,
            pltpu.SemaphoreType.DMA,
            pltpu.SemaphoreType.DMA,
        ],
        compiler_params=pltpu.CompilerParams(needs_layout_passes=False),
        name="sc_main",
    )(row_abs, col_r, ew_r, y_flat)


def _tc_final_kernel(acc_ref, dib_ref, y_ref,
                     Wz_ref, Lz_ref, lbz_ref, bz_ref,
                     Wh_ref, Lh_ref, lbh_ref, bh_ref,
                     Wlin_ref, blin_ref, out_ref, hsum):
    t = pl.program_id(1)
    agg = dib_ref[0] * (acc_ref[0, 0] + acc_ref[1, 0] + y_ref[0])

    Lz1 = Lz_ref[:D, :]
    Lh1 = Lh_ref[:D, :]
    Mz = jnp.dot(Wz_ref[...], Lz1, precision=_HI)
    Mh = jnp.dot(Wh_ref[...], Lh1, precision=_HI)
    cz = jnp.dot(bz_ref[...], Lz1, precision=_HI) + lbz_ref[...]
    ch = jnp.dot(bh_ref[...], Lh1, precision=_HI) + lbh_ref[...]

    Z = jax.nn.sigmoid(jnp.dot(agg, Mz, precision=_HI) + cz)
    Ht = jnp.tanh(jnp.dot(agg, Mh, precision=_HI) + ch)
    Hn = (1.0 - Z) * Ht

    @pl.when(t == 0)
    def _():
        hsum[...] = Hn

    @pl.when(t > 0)
    def _():
        hsum[...] = hsum[...] + Hn

    @pl.when(t == T - 1)
    def _():
        out_ref[...] = (jnp.dot(hsum[...], Wlin_ref[...] * (1.0 / T),
                                precision=_HI) + blin_ref[...])


def _tc_final(acc, dib, y, W_z, L_z, lb_z, b_z, W_h, L_h, lb_h, b_h,
              W_lin, b_lin):
    BN = 1024
    grid = (NPAD // BN, T)

    def full(shape):
        return pl.BlockSpec(shape, lambda i, t: tuple(0 for _ in shape))

    return pl.pallas_call(
        _tc_final_kernel,
        grid=grid,
        in_specs=[
            pl.BlockSpec((2, 1, BN, D), lambda i, t: (0, t, i, 0)),
            pl.BlockSpec((1, BN, D), lambda i, t: (t, i, 0)),
            pl.BlockSpec((1, BN, D), lambda i, t: (t, i, 0)),
            full((D, D)), full((2 * D, D)), full((1, D)), full((1, D)),
            full((D, D)), full((2 * D, D)), full((1, D)), full((1, D)),
            full((D, D)), full((1, D)),
        ],
        out_specs=pl.BlockSpec((BN, D), lambda i, t: (i, 0)),
        out_shape=jax.ShapeDtypeStruct((NPAD, D), jnp.float32),
        scratch_shapes=[pltpu.VMEM((BN, D), jnp.float32)],
        name="tc_final",
    )(acc, dib, y, W_z, L_z, lb_z, b_z, W_h, L_h, lb_h, b_h, W_lin, b_lin)


def kernel(x_seq, edge_index_seq, edge_attr_seq, W_z, b_z, W_r, b_r, W_h, b_h,
           L_z, lb_z, L_r, lb_r, L_h, lb_h, W_lin, b_lin):
    del W_r, b_r, L_r, lb_r

    col_d = edge_index_seq[:, 1, :].reshape(T, NW, NSUB_D, SUB_D, CHUNK_D)
    ew_d = edge_attr_seq[:, :, 0].reshape(T, NW, NSUB_D, SUB_D, CHUNK_D)

    row = edge_index_seq[:, 0, :].reshape(T, NW, NCH, CHUNK)
    col = edge_index_seq[:, 1, :].reshape(T, NW, NSUB, SUB, CHUNK)
    ew = edge_attr_seq[:, :, 0].reshape(T, NW, NSUB, SUB * CHUNK)

    deg_part = _sc_deg(col_d, ew_d)
    x_pad = jnp.pad(x_seq, ((0, 0), (0, NPAD - N), (0, 0)))
    y, dib = _tc_prep(deg_part, x_pad)

    toff = (jnp.arange(T, dtype=jnp.int32) * NPAD).reshape(T, 1, 1, 1)
    row_abs = row + toff
    acc = _sc_main(row_abs, col, ew, y.reshape(T * NPAD, D))

    out = _tc_final(acc, dib, y,
                    W_z, L_z, lb_z.reshape(1, D), b_z.reshape(1, D),
                    W_h, L_h, lb_h.reshape(1, D), b_h.reshape(1, D),
                    W_lin, b_lin.reshape(1, D))
    return out[:N]
